# baseline (speedup 1.0000x reference)
import functools

import jax
import jax.numpy as jnp
from jax import lax
from jax.experimental import pallas as pl
from jax.experimental.pallas import tpu as pltpu
from jax.experimental.pallas import tpu_sc as plsc

NUM_EMB = 1000000
DIM = 64
BATCH = 4096
FIELDS = 26

NC = 2
NS = 16
NW = NC * NS
W_FEAT = DIM // NW

_mesh = plsc.VectorSubcoreMesh(core_axis_name="c", subcore_axis_name="s")


@functools.partial(
    pl.kernel,
    mesh=_mesh,
    out_type=jax.ShapeDtypeStruct((FIELDS, DIM, BATCH), jnp.float32),
    compiler_params=pltpu.CompilerParams(use_tc_tiling_on_sc=False),
    scratch_types=[
        pltpu.VMEM((2, BATCH), jnp.int32),
        pltpu.VMEM((W_FEAT, 2, BATCH), jnp.float32),
        pltpu.SemaphoreType.DMA,
        pltpu.SemaphoreType.DMA,
        pltpu.SemaphoreType.DMA,
    ],
)
def _gather_kernel(xT_hbm, tabT_hbm, outT_hbm, idx_v, rows_v, isem, gsem, wsem):
    wid = lax.axis_index("s") * NC + lax.axis_index("c")
    c0 = wid * W_FEAT

    pltpu.sync_copy(xT_hbm.at[0], idx_v.at[0])
    for p in range(W_FEAT):
        pltpu.async_copy(
            tabT_hbm.at[c0 + p].at[idx_v.at[0]], rows_v.at[p].at[0], gsem)
    pltpu.async_copy(xT_hbm.at[1], idx_v.at[1], isem)

    def body(f, _):
        slot = f % 2
        nslot = (f + 1) % 2
        for p in range(W_FEAT):
            pltpu.make_async_copy(
                xT_hbm.at[0], rows_v.at[p].at[slot], gsem).wait()

        @pl.when(f + 1 < FIELDS)
        def _():
            pltpu.make_async_copy(
                xT_hbm.at[f + 1], idx_v.at[nslot], isem).wait()
            for p in range(W_FEAT):
                pltpu.async_copy(
                    tabT_hbm.at[c0 + p].at[idx_v.at[nslot]],
                    rows_v.at[p].at[nslot], gsem)

        for p in range(W_FEAT):
            pltpu.async_copy(
                rows_v.at[p].at[slot], outT_hbm.at[f, c0 + p], wsem)
        for p in range(W_FEAT):
            pltpu.make_async_copy(
                rows_v.at[p].at[slot], outT_hbm.at[f, c0 + p], wsem).wait()

        @pl.when(f + 2 < FIELDS)
        def _():
            pltpu.async_copy(xT_hbm.at[f + 2], idx_v.at[slot], isem)

        return _

    lax.fori_loop(0, FIELDS, body, None)


def kernel(x, weight):
    xT = x.astype(jnp.int32).T
    wT = weight.T
    outT = _gather_kernel(xT, wT)
    return outT.transpose(2, 0, 1)


# speedup vs baseline: 8.0093x; 8.0093x over previous
import functools

import jax
import jax.numpy as jnp
from jax import lax
from jax.experimental import pallas as pl
from jax.experimental.pallas import tpu as pltpu
from jax.experimental.pallas import tpu_sc as plsc

NUM_EMB = 1000000
DIM = 64
PDIM = 128
BATCH = 4096
FIELDS = 26
TOTAL = BATCH * FIELDS

NC = 2
NS = 16
NW = NC * NS
PER_W = TOTAL // NW
CHUNK = 128
NCHUNK = PER_W // CHUNK
NBUF = 4

_mesh = plsc.VectorSubcoreMesh(core_axis_name="c", subcore_axis_name="s")


@functools.partial(
    pl.kernel,
    mesh=_mesh,
    out_type=jax.ShapeDtypeStruct((TOTAL, PDIM), jnp.float32),
    compiler_params=pltpu.CompilerParams(use_tc_tiling_on_sc=True),
    scratch_types=[
        pltpu.VMEM((PER_W,), jnp.int32),
        pltpu.VMEM((NBUF, CHUNK, PDIM), jnp.float32),
        pltpu.SemaphoreType.DMA,
        pltpu.SemaphoreType.DMA,
    ],
)
def _gather_kernel(idx_hbm, tab_hbm, out_hbm, idx_v, rows_v, gsem, wsem):
    wid = lax.axis_index("s") * NC + lax.axis_index("c")
    base = wid * PER_W

    pltpu.sync_copy(idx_hbm.at[pl.ds(base, PER_W)], idx_v)

    gathers = [None] * NCHUNK
    writes = [None] * NCHUNK
    for j in range(min(NBUF, NCHUNK)):
        gathers[j] = pltpu.async_copy(
            tab_hbm.at[idx_v.at[pl.ds(j * CHUNK, CHUNK)]],
            rows_v.at[j % NBUF], gsem)
    for j in range(NCHUNK):
        gathers[j].wait()
        writes[j] = pltpu.async_copy(
            rows_v.at[j % NBUF],
            out_hbm.at[pl.ds(base + j * CHUNK, CHUNK)],
            wsem)
        nxt = j + NBUF
        if nxt < NCHUNK:
            writes[j].wait()
            gathers[nxt] = pltpu.async_copy(
                tab_hbm.at[idx_v.at[pl.ds(nxt * CHUNK, CHUNK)]],
                rows_v.at[nxt % NBUF], gsem)
    for j in range(max(0, NCHUNK - NBUF), NCHUNK):
        writes[j].wait()


def kernel(x, weight):
    idx = x.astype(jnp.int32).reshape(TOTAL)
    wp = jnp.pad(weight, ((0, 0), (0, PDIM - DIM)))
    out = _gather_kernel(idx, wp)
    return out[:, :DIM].reshape(BATCH, FIELDS, DIM)
